# baseline (speedup 1.0000x reference)
import functools

import jax
import jax.numpy as jnp
from jax import lax
from jax.experimental import pallas as pl
from jax.experimental.pallas import tpu as pltpu
from jax.experimental.pallas import tpu_sc as plsc

N = 10000
P = 128
E = 320000
NC = 2
NS = 16
NW = NC * NS
EPT = E // NW
K = 80
CHUNKS = EPT // K
RPT = N // NS
CE = 2000
CB = 25
BP = 1000


def _prep_body(steps_ref, mu_ref, ps_ref, sel_ref, scaled_ref, f_ref, g_ref):
    st = steps_ref[0, 0]
    ps = ps_ref[...]
    sel = sel_ref[...]
    pst = ps * st
    f = pst * (1.0 - sel)
    f_ref[...] = f
    g_ref[...] = sel + (1.0 - sel) * pst
    scaled_ref[...] = f * mu_ref[...]


def _sc_body(mu_hbm, src_hbm, ew_hbm, f_hbm,
             srcc_hbm, ewc_hbm, g_hbm, dst2_hbm, dstc2_hbm, zrows_hbm,
             g2_out, bp_out, cp_out,
             G_sh, esrc, eew, tbl, acc, rows0, rows1, d2,
             sg0, sg1, ss0, ss1):
    c = lax.axis_index("c")
    s = lax.axis_index("s")
    wid = c * NS + s
    ebase = wid * EPT
    zv = jnp.zeros((16,), jnp.float32)

    def _zg(k, carry):
        j = k * NS + s

        @pl.when(j < N // K)
        def _():
            pltpu.sync_copy(zrows_hbm, G_sh.at[pl.ds(j * K, K)])
        return carry
    lax.fori_loop(0, (N // K + NS - 1) // NS, _zg, 0)
    plsc.subcore_barrier()

    def _zacc(i, carry):
        acc[pl.ds(i * 16, 16)] = zv
        return carry

    bufs = (rows0, rows1)
    gsems = (sg0, sg1)
    ssems = (ss0, ss1)

    pltpu.sync_copy(f_hbm, tbl)
    lax.fori_loop(0, N // 16, _zacc, 0)

    def _outer_rows(jo, carry):
        blk = wid * (EPT // CE) + jo
        off = ebase + jo * CE
        pltpu.sync_copy(src_hbm.at[pl.ds(off, CE)], esrc)
        pltpu.sync_copy(dst2_hbm.at[blk], d2)
        pltpu.sync_copy(ew_hbm.at[pl.ds(off, CE)], eew)
        pend_g = [None, None]
        pend_s = [None, None]
        pend_g[0] = pltpu.async_copy(mu_hbm.at[esrc.at[pl.ds(0, K)]],
                                     bufs[0], gsems[0])
        for ji in range(CB):
            p = ji % 2
            q = 1 - p
            if ji + 1 < CB:
                if pend_s[q] is not None:
                    pend_s[q].wait()
                pend_g[q] = pltpu.async_copy(
                    mu_hbm.at[esrc.at[pl.ds((ji + 1) * K, K)]],
                    bufs[q], gsems[q])
            for l in range(K // 16):
                sl = pl.ds(ji * K + l * 16, 16)
                fv = plsc.load_gather(tbl, [esrc[sl]])
                plsc.addupdate_scatter(acc, [d2[ji, pl.ds(l * 16, 16)]],
                                       fv * eew[sl])
            pend_g[p].wait()
            pend_s[p] = pltpu.async_copy(bufs[p], G_sh.at[d2.at[ji]],
                                         ssems[p], add=True)
        for p in (0, 1):
            if pend_s[p] is not None:
                pend_s[p].wait()
        return carry
    lax.fori_loop(0, EPT // CE, _outer_rows, 0)
    pltpu.sync_copy(acc, bp_out.at[wid])

    pltpu.sync_copy(g_hbm, tbl)
    lax.fori_loop(0, N // 16, _zacc, 0)

    def _outer_c(jo, carry):
        blk = wid * (EPT // CE) + jo
        off = ebase + jo * CE
        pltpu.sync_copy(srcc_hbm.at[pl.ds(off, CE)], esrc)
        pltpu.sync_copy(dstc2_hbm.at[blk], d2)
        pltpu.sync_copy(ewc_hbm.at[pl.ds(off, CE)], eew)

        def _edge(ji, c2):
            for l in range(K // 16):
                sl = pl.ds(ji * K + l * 16, 16)
                gv = plsc.load_gather(tbl, [esrc[sl]])
                plsc.addupdate_scatter(acc, [d2[ji, pl.ds(l * 16, 16)]],
                                       gv * eew[sl])
            return c2
        lax.fori_loop(0, CB, _edge, 0)
        return carry
    lax.fori_loop(0, EPT // CE, _outer_c, 0)
    pltpu.sync_copy(acc, cp_out.at[wid])

    plsc.subcore_barrier()

    def _wg(k, carry):
        j = k * NS + s

        @pl.when(j < N // K)
        def _():
            r0 = j * K
            pltpu.sync_copy(G_sh.at[pl.ds(r0, K)], g2_out.at[c, pl.ds(r0, K)])
        return carry
    lax.fori_loop(0, (N // K + NS - 1) // NS, _wg, 0)


def _final_body(g0_ref, g1_ref, bp_ref, cp_ref, ic_ref, w1_ref, w1c_ref, w2_ref,
                out_ref, wf_s, r1_s, r1c_s):
    @pl.when(pl.program_id(0) == 0)
    def _():
        w1 = w1_ref[...]
        wf_s[...] = jnp.dot(w1, w2_ref[...], preferred_element_type=jnp.float32)
        r1_s[...] = jnp.sum(w1, axis=1)[None, :]
        r1c_s[...] = jnp.sum(w1c_ref[...], axis=1)[None, :]

    G = g0_ref[0] + g1_ref[0]
    b = jnp.sum(bp_ref[:, 0, 0, :], axis=0)[:, None]
    cc = jnp.sum(cp_ref[:, 0, 0, :], axis=0)[:, None]
    inner = lax.dot_general(G, wf_s[...], (((1,), (1,)), ((), ())),
                            preferred_element_type=jnp.float32)
    inner = inner + b * r1_s[...]
    out_ref[...] = (ic_ref[...] + jnp.maximum(inner, 0.0)
                    + jnp.maximum(cc * r1c_s[...], 0.0))


def kernel(old_mu, edge_index, edge_weight, edge_index_complex, edge_weight_complex,
           Ps, IncurredCosts, selected, StepsRemaining, W1, W1c, W2):
    src = edge_index[0]
    dst = edge_index[1]
    srcc = edge_index_complex[0]
    dstc = edge_index_complex[1]
    sel2d = selected.astype(jnp.float32).reshape(N, 1)
    steps2d = jnp.asarray(StepsRemaining, jnp.float32).reshape(1, 1)

    scaled, f_col, g_col = pl.pallas_call(
        _prep_body,
        grid=(N // BP,),
        in_specs=[
            pl.BlockSpec(memory_space=pltpu.SMEM),
            pl.BlockSpec((BP, P), lambda i: (i, 0)),
            pl.BlockSpec((BP, 1), lambda i: (i, 0)),
            pl.BlockSpec((BP, 1), lambda i: (i, 0)),
        ],
        out_specs=[
            pl.BlockSpec((BP, P), lambda i: (i, 0)),
            pl.BlockSpec((BP, 1), lambda i: (i, 0)),
            pl.BlockSpec((BP, 1), lambda i: (i, 0)),
        ],
        out_shape=[
            jax.ShapeDtypeStruct((N, P), jnp.float32),
            jax.ShapeDtypeStruct((N, 1), jnp.float32),
            jax.ShapeDtypeStruct((N, 1), jnp.float32),
        ],
    )(steps2d, old_mu, Ps, sel2d)

    f = f_col.reshape(N)
    g = g_col.reshape(N)

    sc_k = pl.kernel(
        _sc_body,
        mesh=plsc.VectorSubcoreMesh(core_axis_name="c", subcore_axis_name="s"),
        compiler_params=pltpu.CompilerParams(needs_layout_passes=False),
        out_type=[
            jax.ShapeDtypeStruct((NC, N, P), jnp.float32),
            jax.ShapeDtypeStruct((NW, N), jnp.float32),
            jax.ShapeDtypeStruct((NW, N), jnp.float32),
        ],
        scratch_types=[
            pltpu.VMEM_SHARED((N, P), jnp.float32),
            pltpu.VMEM((CE,), jnp.int32),
            pltpu.VMEM((CE,), jnp.float32),
            pltpu.VMEM((N,), jnp.float32),
            pltpu.VMEM((N,), jnp.float32),
            pltpu.VMEM((K, P), jnp.float32),
            pltpu.VMEM((K, P), jnp.float32),
            pltpu.VMEM((CB, K), jnp.int32),
            pltpu.SemaphoreType.DMA,
            pltpu.SemaphoreType.DMA,
            pltpu.SemaphoreType.DMA,
            pltpu.SemaphoreType.DMA,
        ],
    )
    G2, bpart, cpart = sc_k(scaled, src, edge_weight, f,
                            srcc, edge_weight_complex, g,
                            dst.reshape(E // CE, CB, K),
                            dstc.reshape(E // CE, CB, K),
                            jnp.zeros((K, P), jnp.float32))

    mu = pl.pallas_call(
        _final_body,
        grid=(N // BP,),
        in_specs=[
            pl.BlockSpec((1, BP, P), lambda i: (0, i, 0)),
            pl.BlockSpec((1, BP, P), lambda i: (1, i, 0)),
            pl.BlockSpec((NW, 1, 1, BP), lambda i: (0, i, 0, 0)),
            pl.BlockSpec((NW, 1, 1, BP), lambda i: (0, i, 0, 0)),
            pl.BlockSpec((BP, 1), lambda i: (i, 0)),
            pl.BlockSpec((P, P), lambda i: (0, 0)),
            pl.BlockSpec((P, P), lambda i: (0, 0)),
            pl.BlockSpec((P, P), lambda i: (0, 0)),
        ],
        out_specs=pl.BlockSpec((BP, P), lambda i: (i, 0)),
        out_shape=jax.ShapeDtypeStruct((N, P), jnp.float32),
        scratch_shapes=[
            pltpu.VMEM((P, P), jnp.float32),
            pltpu.VMEM((1, P), jnp.float32),
            pltpu.VMEM((1, P), jnp.float32),
        ],
    )(G2, G2,
      bpart.reshape(NW, N // BP, 1, BP),
      cpart.reshape(NW, N // BP, 1, BP),
      IncurredCosts, W1, W1c, W2)
    return mu


# speedup vs baseline: 1.1093x; 1.1093x over previous
import functools

import jax
import jax.numpy as jnp
from jax import lax
from jax.experimental import pallas as pl
from jax.experimental.pallas import tpu as pltpu
from jax.experimental.pallas import tpu_sc as plsc

N = 10000
P = 128
E = 320000
NC = 2
NS = 16
NW = NC * NS
EPT = E // NW
K = 80
CHUNKS = EPT // K
RPT = N // NS
CE = 2000
CB = 25
BP = 1000


def _prep_body(steps_ref, mu_ref, ps_ref, sel_ref, scaled_ref, f_ref, g_ref):
    st = steps_ref[0, 0]
    ps = ps_ref[...]
    sel = sel_ref[...]
    pst = ps * st
    f = pst * (1.0 - sel)
    f_ref[...] = f
    g_ref[...] = sel + (1.0 - sel) * pst
    scaled_ref[...] = f * mu_ref[...]


def _sc_body(mu_hbm, src_hbm, ew_hbm, f_hbm,
             srcc_hbm, ewc_hbm, g_hbm, dst2_hbm, dstc2_hbm,
             g2_out, bp_out, cp_out,
             G_sh, esrc, eew, tbl, acc, rows0, rows1, d2,
             sg0, sg1, ss0, ss1):
    c = lax.axis_index("c")
    s = lax.axis_index("s")
    wid = c * NS + s
    ebase = wid * EPT
    zv = jnp.zeros((16,), jnp.float32)

    def _zrow(i, carry):
        rows0[i // 8, pl.ds((i % 8) * 16, 16)] = zv
        return carry
    lax.fori_loop(0, K * 8, _zrow, 0)

    def _zg(k, carry):
        j = k * NS + s

        @pl.when(j < N // K)
        def _():
            pltpu.sync_copy(rows0, G_sh.at[pl.ds(j * K, K)])
        return carry
    lax.fori_loop(0, (N // K + NS - 1) // NS, _zg, 0)
    plsc.subcore_barrier()

    def _zacc(i, carry):
        acc[pl.ds(i * 16, 16)] = zv
        return carry

    bufs = (rows0, rows1)
    gsems = (sg0, sg1)
    ssems = (ss0, ss1)

    pltpu.sync_copy(f_hbm, tbl)
    lax.fori_loop(0, N // 16, _zacc, 0)

    def _outer_rows(jo, carry):
        blk = wid * (EPT // CE) + jo
        off = ebase + jo * CE
        h1 = pltpu.async_copy(src_hbm.at[pl.ds(off, CE)], esrc, sg0)
        h2 = pltpu.async_copy(dst2_hbm.at[blk], d2, sg1)
        h3 = pltpu.async_copy(ew_hbm.at[pl.ds(off, CE)], eew, ss0)
        h1.wait()
        h2.wait()
        h3.wait()
        pend_g = [None, None]
        pend_s = [None, None]
        pend_g[0] = pltpu.async_copy(mu_hbm.at[esrc.at[pl.ds(0, K)]],
                                     bufs[0], gsems[0])
        for ji in range(CB):
            p = ji % 2
            q = 1 - p
            if ji + 1 < CB:
                if pend_s[q] is not None:
                    pend_s[q].wait()
                pend_g[q] = pltpu.async_copy(
                    mu_hbm.at[esrc.at[pl.ds((ji + 1) * K, K)]],
                    bufs[q], gsems[q])
            for l in range(K // 16):
                sl = pl.ds(ji * K + l * 16, 16)
                fv = plsc.load_gather(tbl, [esrc[sl]])
                plsc.addupdate_scatter(acc, [d2[ji, pl.ds(l * 16, 16)]],
                                       fv * eew[sl])
            pend_g[p].wait()
            pend_s[p] = pltpu.async_copy(bufs[p], G_sh.at[d2.at[ji]],
                                         ssems[p], add=True)
        for p in (0, 1):
            if pend_s[p] is not None:
                pend_s[p].wait()
        return carry
    lax.fori_loop(0, EPT // CE, _outer_rows, 0)
    pltpu.sync_copy(acc, bp_out.at[wid])

    pltpu.sync_copy(g_hbm, tbl)
    lax.fori_loop(0, N // 16, _zacc, 0)

    def _outer_c(jo, carry):
        blk = wid * (EPT // CE) + jo
        off = ebase + jo * CE
        h1 = pltpu.async_copy(srcc_hbm.at[pl.ds(off, CE)], esrc, sg0)
        h2 = pltpu.async_copy(dstc2_hbm.at[blk], d2, sg1)
        h3 = pltpu.async_copy(ewc_hbm.at[pl.ds(off, CE)], eew, ss0)
        h1.wait()
        h2.wait()
        h3.wait()

        def _edge(ji, c2):
            for l in range(K // 16):
                sl = pl.ds(ji * K + l * 16, 16)
                gv = plsc.load_gather(tbl, [esrc[sl]])
                plsc.addupdate_scatter(acc, [d2[ji, pl.ds(l * 16, 16)]],
                                       gv * eew[sl])
            return c2
        lax.fori_loop(0, CB, _edge, 0)
        return carry
    lax.fori_loop(0, EPT // CE, _outer_c, 0)
    pltpu.sync_copy(acc, cp_out.at[wid])

    plsc.subcore_barrier()

    def _wg(k, carry):
        j = k * NS + s

        @pl.when(j < N // K)
        def _():
            r0 = j * K
            pltpu.sync_copy(G_sh.at[pl.ds(r0, K)], g2_out.at[c, pl.ds(r0, K)])
        return carry
    lax.fori_loop(0, (N // K + NS - 1) // NS, _wg, 0)


def _final_body(g0_ref, g1_ref, bp_ref, cp_ref, ic_ref, w1_ref, w1c_ref, w2_ref,
                out_ref, wf_s, r1_s, r1c_s):
    @pl.when(pl.program_id(0) == 0)
    def _():
        w1 = w1_ref[...]
        wf_s[...] = jnp.dot(w1, w2_ref[...], preferred_element_type=jnp.float32)
        r1_s[...] = jnp.sum(w1, axis=1)[None, :]
        r1c_s[...] = jnp.sum(w1c_ref[...], axis=1)[None, :]

    G = g0_ref[0] + g1_ref[0]
    b = jnp.sum(bp_ref[:, 0, 0, :], axis=0)[:, None]
    cc = jnp.sum(cp_ref[:, 0, 0, :], axis=0)[:, None]
    inner = lax.dot_general(G, wf_s[...], (((1,), (1,)), ((), ())),
                            preferred_element_type=jnp.float32)
    inner = inner + b * r1_s[...]
    out_ref[...] = (ic_ref[...] + jnp.maximum(inner, 0.0)
                    + jnp.maximum(cc * r1c_s[...], 0.0))


def kernel(old_mu, edge_index, edge_weight, edge_index_complex, edge_weight_complex,
           Ps, IncurredCosts, selected, StepsRemaining, W1, W1c, W2):
    src = edge_index[0]
    dst = edge_index[1]
    srcc = edge_index_complex[0]
    dstc = edge_index_complex[1]
    sel2d = selected.astype(jnp.float32).reshape(N, 1)
    steps2d = jnp.asarray(StepsRemaining, jnp.float32).reshape(1, 1)

    scaled, f_col, g_col = pl.pallas_call(
        _prep_body,
        grid=(N // BP,),
        in_specs=[
            pl.BlockSpec(memory_space=pltpu.SMEM),
            pl.BlockSpec((BP, P), lambda i: (i, 0)),
            pl.BlockSpec((BP, 1), lambda i: (i, 0)),
            pl.BlockSpec((BP, 1), lambda i: (i, 0)),
        ],
        out_specs=[
            pl.BlockSpec((BP, P), lambda i: (i, 0)),
            pl.BlockSpec((BP, 1), lambda i: (i, 0)),
            pl.BlockSpec((BP, 1), lambda i: (i, 0)),
        ],
        out_shape=[
            jax.ShapeDtypeStruct((N, P), jnp.float32),
            jax.ShapeDtypeStruct((N, 1), jnp.float32),
            jax.ShapeDtypeStruct((N, 1), jnp.float32),
        ],
    )(steps2d, old_mu, Ps, sel2d)

    f = f_col.reshape(N)
    g = g_col.reshape(N)

    sc_k = pl.kernel(
        _sc_body,
        mesh=plsc.VectorSubcoreMesh(core_axis_name="c", subcore_axis_name="s"),
        compiler_params=pltpu.CompilerParams(needs_layout_passes=False),
        out_type=[
            jax.ShapeDtypeStruct((NC, N, P), jnp.float32),
            jax.ShapeDtypeStruct((NW, N), jnp.float32),
            jax.ShapeDtypeStruct((NW, N), jnp.float32),
        ],
        scratch_types=[
            pltpu.VMEM_SHARED((N, P), jnp.float32),
            pltpu.VMEM((CE,), jnp.int32),
            pltpu.VMEM((CE,), jnp.float32),
            pltpu.VMEM((N,), jnp.float32),
            pltpu.VMEM((N,), jnp.float32),
            pltpu.VMEM((K, P), jnp.float32),
            pltpu.VMEM((K, P), jnp.float32),
            pltpu.VMEM((CB, K), jnp.int32),
            pltpu.SemaphoreType.DMA,
            pltpu.SemaphoreType.DMA,
            pltpu.SemaphoreType.DMA,
            pltpu.SemaphoreType.DMA,
        ],
    )
    G2, bpart, cpart = sc_k(scaled, src, edge_weight, f,
                            srcc, edge_weight_complex, g,
                            dst.reshape(E // CE, CB, K),
                            dstc.reshape(E // CE, CB, K))

    mu = pl.pallas_call(
        _final_body,
        grid=(N // BP,),
        in_specs=[
            pl.BlockSpec((1, BP, P), lambda i: (0, i, 0)),
            pl.BlockSpec((1, BP, P), lambda i: (1, i, 0)),
            pl.BlockSpec((NW, 1, 1, BP), lambda i: (0, i, 0, 0)),
            pl.BlockSpec((NW, 1, 1, BP), lambda i: (0, i, 0, 0)),
            pl.BlockSpec((BP, 1), lambda i: (i, 0)),
            pl.BlockSpec((P, P), lambda i: (0, 0)),
            pl.BlockSpec((P, P), lambda i: (0, 0)),
            pl.BlockSpec((P, P), lambda i: (0, 0)),
        ],
        out_specs=pl.BlockSpec((BP, P), lambda i: (i, 0)),
        out_shape=jax.ShapeDtypeStruct((N, P), jnp.float32),
        scratch_shapes=[
            pltpu.VMEM((P, P), jnp.float32),
            pltpu.VMEM((1, P), jnp.float32),
            pltpu.VMEM((1, P), jnp.float32),
        ],
    )(G2, G2,
      bpart.reshape(NW, N // BP, 1, BP),
      cpart.reshape(NW, N // BP, 1, BP),
      IncurredCosts, W1, W1c, W2)
    return mu


# speedup vs baseline: 1.1140x; 1.0043x over previous
import functools

import jax
import jax.numpy as jnp
from jax import lax
from jax.experimental import pallas as pl
from jax.experimental.pallas import tpu as pltpu
from jax.experimental.pallas import tpu_sc as plsc

N = 10000
P = 128
E = 320000
NC = 2
NS = 16
NW = NC * NS
EPT = E // NW
K = 80
CHUNKS = EPT // K
RPT = N // NS
CE = 2000
CB = 25
BP = 1000


def _prep_body(steps_ref, mu_ref, ps_ref, sel_ref, scaled_ref, f_ref, g_ref):
    st = steps_ref[0, 0]
    ps = ps_ref[...]
    sel = sel_ref[...]
    pst = ps * st
    f = pst * (1.0 - sel)
    f_ref[...] = f
    g_ref[...] = sel + (1.0 - sel) * pst
    scaled_ref[...] = f * mu_ref[...]


def _sc_body(mu_hbm, src_hbm, ew_hbm, f_hbm,
             srcc_hbm, ewc_hbm, g_hbm, dst2_hbm, dstc2_hbm,
             g2_out, bp_out, cp_out,
             G_sh, esrc, eew, tbl, acc, rows0, rows1, d2,
             sg0, sg1, ss0, ss1):
    c = lax.axis_index("c")
    s = lax.axis_index("s")
    wid = c * NS + s
    ebase = wid * EPT
    zv = jnp.zeros((16,), jnp.float32)

    def _zrow(i, carry):
        rows0[i // 8, pl.ds((i % 8) * 16, 16)] = zv
        return carry
    lax.fori_loop(0, K * 8, _zrow, 0)

    def _zg(k, carry):
        j = k * NS + s

        @pl.when(j < N // K)
        def _():
            pltpu.sync_copy(rows0, G_sh.at[pl.ds(j * K, K)])
        return carry
    lax.fori_loop(0, (N // K + NS - 1) // NS, _zg, 0)
    plsc.subcore_barrier()

    def _zacc(i, carry):
        acc[pl.ds(i * 16, 16)] = zv
        return carry

    bufs = (rows0, rows1)
    gsems = (sg0, sg1)
    ssems = (ss0, ss1)

    pltpu.sync_copy(f_hbm, tbl)
    lax.fori_loop(0, N // 16, _zacc, 0)

    def _outer_rows(jo, carry):
        blk = wid * (EPT // CE) + jo
        off = ebase + jo * CE
        h1 = pltpu.async_copy(src_hbm.at[pl.ds(off, CE)], esrc, sg0)
        h2 = pltpu.async_copy(dst2_hbm.at[blk], d2, sg1)
        h3 = pltpu.async_copy(ew_hbm.at[pl.ds(off, CE)], eew, ss0)
        h1.wait()
        pend_g = [None, None]
        pend_s = [None, None]
        pend_g[0] = pltpu.async_copy(mu_hbm.at[esrc.at[pl.ds(0, K)]],
                                     bufs[0], gsems[0])
        h2.wait()
        h3.wait()
        for ji in range(CB):
            p = ji % 2
            q = 1 - p
            if ji + 1 < CB:
                if pend_s[q] is not None:
                    pend_s[q].wait()
                pend_g[q] = pltpu.async_copy(
                    mu_hbm.at[esrc.at[pl.ds((ji + 1) * K, K)]],
                    bufs[q], gsems[q])
            for l in range(K // 16):
                sl = pl.ds(ji * K + l * 16, 16)
                fv = plsc.load_gather(tbl, [esrc[sl]])
                plsc.addupdate_scatter(acc, [d2[ji, pl.ds(l * 16, 16)]],
                                       fv * eew[sl])
            pend_g[p].wait()
            pend_s[p] = pltpu.async_copy(bufs[p], G_sh.at[d2.at[ji]],
                                         ssems[p], add=True)
        for p in (0, 1):
            if pend_s[p] is not None:
                pend_s[p].wait()
        return carry
    lax.fori_loop(0, EPT // CE, _outer_rows, 0)
    pltpu.sync_copy(acc, bp_out.at[wid])

    plsc.subcore_barrier()

    NWG = (N // K + NS - 1) // NS
    for k in range(NWG):
        j = k * NS + s

        @pl.when(j < N // K)
        def _():
            pltpu.async_copy(G_sh.at[pl.ds(j * K, K)],
                             g2_out.at[c, pl.ds(j * K, K)], ss1)

    pltpu.sync_copy(g_hbm, tbl)
    lax.fori_loop(0, N // 16, _zacc, 0)

    def _outer_c(jo, carry):
        blk = wid * (EPT // CE) + jo
        off = ebase + jo * CE
        h1 = pltpu.async_copy(srcc_hbm.at[pl.ds(off, CE)], esrc, sg0)
        h2 = pltpu.async_copy(dstc2_hbm.at[blk], d2, sg1)
        h3 = pltpu.async_copy(ewc_hbm.at[pl.ds(off, CE)], eew, ss0)
        h1.wait()
        h2.wait()
        h3.wait()

        def _edge(ji, c2):
            for l in range(K // 16):
                sl = pl.ds(ji * K + l * 16, 16)
                gv = plsc.load_gather(tbl, [esrc[sl]])
                plsc.addupdate_scatter(acc, [d2[ji, pl.ds(l * 16, 16)]],
                                       gv * eew[sl])
            return c2
        lax.fori_loop(0, CB, _edge, 0)
        return carry
    lax.fori_loop(0, EPT // CE, _outer_c, 0)
    pltpu.sync_copy(acc, cp_out.at[wid])

    for k in range(NWG):
        j = k * NS + s

        @pl.when(j < N // K)
        def _():
            pltpu.make_async_copy(G_sh.at[pl.ds(j * K, K)],
                                  g2_out.at[c, pl.ds(j * K, K)], ss1).wait()


def _final_body(g0_ref, g1_ref, bp_ref, cp_ref, ic_ref, w1_ref, w1c_ref, w2_ref,
                out_ref, wf_s, r1_s, r1c_s):
    @pl.when(pl.program_id(0) == 0)
    def _():
        w1 = w1_ref[...]
        wf_s[...] = jnp.dot(w1, w2_ref[...], preferred_element_type=jnp.float32)
        r1_s[...] = jnp.sum(w1, axis=1)[None, :]
        r1c_s[...] = jnp.sum(w1c_ref[...], axis=1)[None, :]

    G = g0_ref[0] + g1_ref[0]
    b = jnp.sum(bp_ref[:, 0, 0, :], axis=0)[:, None]
    cc = jnp.sum(cp_ref[:, 0, 0, :], axis=0)[:, None]
    inner = lax.dot_general(G, wf_s[...], (((1,), (1,)), ((), ())),
                            preferred_element_type=jnp.float32)
    inner = inner + b * r1_s[...]
    out_ref[...] = (ic_ref[...] + jnp.maximum(inner, 0.0)
                    + jnp.maximum(cc * r1c_s[...], 0.0))


def kernel(old_mu, edge_index, edge_weight, edge_index_complex, edge_weight_complex,
           Ps, IncurredCosts, selected, StepsRemaining, W1, W1c, W2):
    src = edge_index[0]
    dst = edge_index[1]
    srcc = edge_index_complex[0]
    dstc = edge_index_complex[1]
    sel2d = selected.astype(jnp.float32).reshape(N, 1)
    steps2d = jnp.asarray(StepsRemaining, jnp.float32).reshape(1, 1)

    scaled, f_col, g_col = pl.pallas_call(
        _prep_body,
        grid=(N // BP,),
        in_specs=[
            pl.BlockSpec(memory_space=pltpu.SMEM),
            pl.BlockSpec((BP, P), lambda i: (i, 0)),
            pl.BlockSpec((BP, 1), lambda i: (i, 0)),
            pl.BlockSpec((BP, 1), lambda i: (i, 0)),
        ],
        out_specs=[
            pl.BlockSpec((BP, P), lambda i: (i, 0)),
            pl.BlockSpec((BP, 1), lambda i: (i, 0)),
            pl.BlockSpec((BP, 1), lambda i: (i, 0)),
        ],
        out_shape=[
            jax.ShapeDtypeStruct((N, P), jnp.float32),
            jax.ShapeDtypeStruct((N, 1), jnp.float32),
            jax.ShapeDtypeStruct((N, 1), jnp.float32),
        ],
    )(steps2d, old_mu, Ps, sel2d)

    f = f_col.reshape(N)
    g = g_col.reshape(N)

    sc_k = pl.kernel(
        _sc_body,
        mesh=plsc.VectorSubcoreMesh(core_axis_name="c", subcore_axis_name="s"),
        compiler_params=pltpu.CompilerParams(needs_layout_passes=False),
        out_type=[
            jax.ShapeDtypeStruct((NC, N, P), jnp.float32),
            jax.ShapeDtypeStruct((NW, N), jnp.float32),
            jax.ShapeDtypeStruct((NW, N), jnp.float32),
        ],
        scratch_types=[
            pltpu.VMEM_SHARED((N, P), jnp.float32),
            pltpu.VMEM((CE,), jnp.int32),
            pltpu.VMEM((CE,), jnp.float32),
            pltpu.VMEM((N,), jnp.float32),
            pltpu.VMEM((N,), jnp.float32),
            pltpu.VMEM((K, P), jnp.float32),
            pltpu.VMEM((K, P), jnp.float32),
            pltpu.VMEM((CB, K), jnp.int32),
            pltpu.SemaphoreType.DMA,
            pltpu.SemaphoreType.DMA,
            pltpu.SemaphoreType.DMA,
            pltpu.SemaphoreType.DMA,
        ],
    )
    G2, bpart, cpart = sc_k(scaled, src, edge_weight, f,
                            srcc, edge_weight_complex, g,
                            dst.reshape(E // CE, CB, K),
                            dstc.reshape(E // CE, CB, K))

    mu = pl.pallas_call(
        _final_body,
        grid=(N // BP,),
        in_specs=[
            pl.BlockSpec((1, BP, P), lambda i: (0, i, 0)),
            pl.BlockSpec((1, BP, P), lambda i: (1, i, 0)),
            pl.BlockSpec((NW, 1, 1, BP), lambda i: (0, i, 0, 0)),
            pl.BlockSpec((NW, 1, 1, BP), lambda i: (0, i, 0, 0)),
            pl.BlockSpec((BP, 1), lambda i: (i, 0)),
            pl.BlockSpec((P, P), lambda i: (0, 0)),
            pl.BlockSpec((P, P), lambda i: (0, 0)),
            pl.BlockSpec((P, P), lambda i: (0, 0)),
        ],
        out_specs=pl.BlockSpec((BP, P), lambda i: (i, 0)),
        out_shape=jax.ShapeDtypeStruct((N, P), jnp.float32),
        scratch_shapes=[
            pltpu.VMEM((P, P), jnp.float32),
            pltpu.VMEM((1, P), jnp.float32),
            pltpu.VMEM((1, P), jnp.float32),
        ],
    )(G2, G2,
      bpart.reshape(NW, N // BP, 1, BP),
      cpart.reshape(NW, N // BP, 1, BP),
      IncurredCosts, W1, W1c, W2)
    return mu


# speedup vs baseline: 1.1283x; 1.0129x over previous
import functools

import jax
import jax.numpy as jnp
from jax import lax
from jax.experimental import pallas as pl
from jax.experimental.pallas import tpu as pltpu
from jax.experimental.pallas import tpu_sc as plsc

N = 10000
P = 128
E = 320000
NC = 2
NS = 16
NW = NC * NS
EPT = E // NW
K = 80
CHUNKS = EPT // K
RPT = N // NS
CE = 2000
CB = 25
BP = 1000


def _prep_body(steps_ref, mu_ref, ps_ref, sel_ref, scaled_ref, f_ref, g_ref):
    st = steps_ref[0, 0]
    ps = ps_ref[...]
    sel = sel_ref[...]
    pst = ps * st
    f = pst * (1.0 - sel)
    f_ref[...] = f
    g_ref[...] = sel + (1.0 - sel) * pst
    scaled_ref[...] = f * mu_ref[...]


def _sc_body(mu_hbm, src_hbm, ew_hbm, f_hbm,
             srcc_hbm, ewc_hbm, g_hbm, dst2_hbm, dstc2_hbm,
             g2_out, bp_out, cp_out,
             G_sh, esrc, eew, tbl, acc, rows0, rows1, d2,
             sg0, sg1, ss0, ss1):
    c = lax.axis_index("c")
    s = lax.axis_index("s")
    wid = c * NS + s
    ebase = wid * EPT
    zv = jnp.zeros((16,), jnp.float32)

    def _zrow(i, carry):
        rows0[i // 8, pl.ds((i % 8) * 16, 16)] = zv
        return carry
    lax.fori_loop(0, K * 8, _zrow, 0)

    def _zg(k, carry):
        j = k * NS + s

        @pl.when(j < N // K)
        def _():
            pltpu.sync_copy(rows0, G_sh.at[pl.ds(j * K, K)])
        return carry
    def _zacc(i, carry):
        acc[pl.ds(i * 16, 16)] = zv
        return carry

    bufs = (rows0, rows1)
    gsems = (sg0, sg1)
    ssems = (ss0, ss1)

    pltpu.sync_copy(f_hbm, tbl)
    lax.fori_loop(0, N // 16, _zacc, 0)
    plsc.subcore_barrier()

    def _outer_rows(jo, carry):
        blk = wid * (EPT // CE) + jo
        off = ebase + jo * CE
        h1 = pltpu.async_copy(src_hbm.at[pl.ds(off, CE)], esrc, sg0)
        h2 = pltpu.async_copy(dst2_hbm.at[blk], d2, sg1)
        h3 = pltpu.async_copy(ew_hbm.at[pl.ds(off, CE)], eew, ss0)
        h1.wait()
        pend_g = [None, None]
        pend_s = [None, None]
        pend_g[0] = pltpu.async_copy(mu_hbm.at[esrc.at[pl.ds(0, K)]],
                                     bufs[0], gsems[0])
        h2.wait()
        h3.wait()
        for ji in range(CB):
            p = ji % 2
            q = 1 - p
            if ji + 1 < CB:
                if pend_s[q] is not None:
                    pend_s[q].wait()
                pend_g[q] = pltpu.async_copy(
                    mu_hbm.at[esrc.at[pl.ds((ji + 1) * K, K)]],
                    bufs[q], gsems[q])
            pend_g[p].wait()
            pend_s[p] = pltpu.async_copy(bufs[p], G_sh.at[d2.at[ji]],
                                         ssems[p], add=True)
            for l in range(K // 16):
                sl = pl.ds(ji * K + l * 16, 16)
                fv = plsc.load_gather(tbl, [esrc[sl]])
                plsc.addupdate_scatter(acc, [d2[ji, pl.ds(l * 16, 16)]],
                                       fv * eew[sl])
        for p in (0, 1):
            if pend_s[p] is not None:
                pend_s[p].wait()
        return carry
    lax.fori_loop(0, EPT // CE, _outer_rows, 0)
    pltpu.sync_copy(acc, bp_out.at[wid])

    plsc.subcore_barrier()

    NWG = (N // K + NS - 1) // NS
    for k in range(NWG):
        j = k * NS + s

        @pl.when(j < N // K)
        def _():
            pltpu.async_copy(G_sh.at[pl.ds(j * K, K)],
                             g2_out.at[c, pl.ds(j * K, K)], ss1)

    pltpu.sync_copy(g_hbm, tbl)
    lax.fori_loop(0, N // 16, _zacc, 0)

    def _outer_c(jo, carry):
        blk = wid * (EPT // CE) + jo
        off = ebase + jo * CE
        h1 = pltpu.async_copy(srcc_hbm.at[pl.ds(off, CE)], esrc, sg0)
        h2 = pltpu.async_copy(dstc2_hbm.at[blk], d2, sg1)
        h3 = pltpu.async_copy(ewc_hbm.at[pl.ds(off, CE)], eew, ss0)
        h1.wait()
        h2.wait()
        h3.wait()

        def _edge(ji, c2):
            for l in range(K // 16):
                sl = pl.ds(ji * K + l * 16, 16)
                gv = plsc.load_gather(tbl, [esrc[sl]])
                plsc.addupdate_scatter(acc, [d2[ji, pl.ds(l * 16, 16)]],
                                       gv * eew[sl])
            return c2
        lax.fori_loop(0, CB, _edge, 0)
        return carry
    lax.fori_loop(0, EPT // CE, _outer_c, 0)
    pltpu.sync_copy(acc, cp_out.at[wid])

    for k in range(NWG):
        j = k * NS + s

        @pl.when(j < N // K)
        def _():
            pltpu.make_async_copy(G_sh.at[pl.ds(j * K, K)],
                                  g2_out.at[c, pl.ds(j * K, K)], ss1).wait()


def _final_body(g0_ref, g1_ref, bp_ref, cp_ref, ic_ref, w1_ref, w1c_ref, w2_ref,
                out_ref, wf_s, r1_s, r1c_s):
    @pl.when(pl.program_id(0) == 0)
    def _():
        w1 = w1_ref[...]
        wf_s[...] = jnp.dot(w1, w2_ref[...], preferred_element_type=jnp.float32)
        r1_s[...] = jnp.sum(w1, axis=1)[None, :]
        r1c_s[...] = jnp.sum(w1c_ref[...], axis=1)[None, :]

    G = g0_ref[0] + g1_ref[0]
    b = jnp.sum(bp_ref[:, 0, 0, :], axis=0)[:, None]
    cc = jnp.sum(cp_ref[:, 0, 0, :], axis=0)[:, None]
    inner = lax.dot_general(G, wf_s[...], (((1,), (1,)), ((), ())),
                            preferred_element_type=jnp.float32)
    inner = inner + b * r1_s[...]
    out_ref[...] = (ic_ref[...] + jnp.maximum(inner, 0.0)
                    + jnp.maximum(cc * r1c_s[...], 0.0))


def kernel(old_mu, edge_index, edge_weight, edge_index_complex, edge_weight_complex,
           Ps, IncurredCosts, selected, StepsRemaining, W1, W1c, W2):
    src = edge_index[0]
    dst = edge_index[1]
    srcc = edge_index_complex[0]
    dstc = edge_index_complex[1]
    sel2d = selected.astype(jnp.float32).reshape(N, 1)
    steps2d = jnp.asarray(StepsRemaining, jnp.float32).reshape(1, 1)

    scaled, f_col, g_col = pl.pallas_call(
        _prep_body,
        grid=(N // BP,),
        in_specs=[
            pl.BlockSpec(memory_space=pltpu.SMEM),
            pl.BlockSpec((BP, P), lambda i: (i, 0)),
            pl.BlockSpec((BP, 1), lambda i: (i, 0)),
            pl.BlockSpec((BP, 1), lambda i: (i, 0)),
        ],
        out_specs=[
            pl.BlockSpec((BP, P), lambda i: (i, 0)),
            pl.BlockSpec((BP, 1), lambda i: (i, 0)),
            pl.BlockSpec((BP, 1), lambda i: (i, 0)),
        ],
        out_shape=[
            jax.ShapeDtypeStruct((N, P), jnp.float32),
            jax.ShapeDtypeStruct((N, 1), jnp.float32),
            jax.ShapeDtypeStruct((N, 1), jnp.float32),
        ],
    )(steps2d, old_mu, Ps, sel2d)

    f = f_col.reshape(N)
    g = g_col.reshape(N)

    sc_k = pl.kernel(
        _sc_body,
        mesh=plsc.VectorSubcoreMesh(core_axis_name="c", subcore_axis_name="s"),
        compiler_params=pltpu.CompilerParams(needs_layout_passes=False),
        out_type=[
            jax.ShapeDtypeStruct((NC, N, P), jnp.float32),
            jax.ShapeDtypeStruct((NW, N), jnp.float32),
            jax.ShapeDtypeStruct((NW, N), jnp.float32),
        ],
        scratch_types=[
            pltpu.VMEM_SHARED((N, P), jnp.float32),
            pltpu.VMEM((CE,), jnp.int32),
            pltpu.VMEM((CE,), jnp.float32),
            pltpu.VMEM((N,), jnp.float32),
            pltpu.VMEM((N,), jnp.float32),
            pltpu.VMEM((K, P), jnp.float32),
            pltpu.VMEM((K, P), jnp.float32),
            pltpu.VMEM((CB, K), jnp.int32),
            pltpu.SemaphoreType.DMA,
            pltpu.SemaphoreType.DMA,
            pltpu.SemaphoreType.DMA,
            pltpu.SemaphoreType.DMA,
        ],
    )
    G2, bpart, cpart = sc_k(scaled, src, edge_weight, f,
                            srcc, edge_weight_complex, g,
                            dst.reshape(E // CE, CB, K),
                            dstc.reshape(E // CE, CB, K))

    mu = pl.pallas_call(
        _final_body,
        grid=(N // BP,),
        in_specs=[
            pl.BlockSpec((1, BP, P), lambda i: (0, i, 0)),
            pl.BlockSpec((1, BP, P), lambda i: (1, i, 0)),
            pl.BlockSpec((NW, 1, 1, BP), lambda i: (0, i, 0, 0)),
            pl.BlockSpec((NW, 1, 1, BP), lambda i: (0, i, 0, 0)),
            pl.BlockSpec((BP, 1), lambda i: (i, 0)),
            pl.BlockSpec((P, P), lambda i: (0, 0)),
            pl.BlockSpec((P, P), lambda i: (0, 0)),
            pl.BlockSpec((P, P), lambda i: (0, 0)),
        ],
        out_specs=pl.BlockSpec((BP, P), lambda i: (i, 0)),
        out_shape=jax.ShapeDtypeStruct((N, P), jnp.float32),
        scratch_shapes=[
            pltpu.VMEM((P, P), jnp.float32),
            pltpu.VMEM((1, P), jnp.float32),
            pltpu.VMEM((1, P), jnp.float32),
        ],
    )(G2, G2,
      bpart.reshape(NW, N // BP, 1, BP),
      cpart.reshape(NW, N // BP, 1, BP),
      IncurredCosts, W1, W1c, W2)
    return mu
